# Initial kernel scaffold; baseline (speedup 1.0000x reference)
import jax
import jax.numpy as jnp
from jax.experimental import pallas as pl


def kernel(h, e, edge_index, edge_w, edge_b, root, bias):
    raise NotImplementedError("write your pallas kernel here")


# speedup vs baseline: 1.5438x; 1.5438x over previous
import functools

import jax
import jax.numpy as jnp
import numpy as np
from jax import lax
from jax.experimental import pallas as pl
from jax.experimental.pallas import tpu as pltpu
from jax.experimental.pallas import tpu_sc as plsc

N_NODES = 10000
N_EDGES = 160000
C = 16

NC = 2
NS = 16
NW = NC * NS
CHUNK = 100
EPW = N_EDGES // NW
NCH = EPW // CHUNK
NCHT = N_EDGES // CHUNK
ROWS_PER_TILE = N_NODES // NS

_REP = np.repeat(np.eye(C, dtype=np.float32), C, axis=1)
_SUM = np.tile(np.eye(C, dtype=np.float32), (C, 1))

_mesh = plsc.VectorSubcoreMesh(core_axis_name="c", subcore_axis_name="s")
_sc_params = pltpu.CompilerParams(use_tc_tiling_on_sc=False)


@functools.partial(
    pl.kernel,
    out_type=jax.ShapeDtypeStruct((NW, NCH, CHUNK, C), jnp.float32),
    mesh=_mesh,
    scratch_types=[
        pltpu.VMEM((NCH, CHUNK), jnp.int32),
        pltpu.VMEM((NCH, CHUNK, C), jnp.float32),
        pltpu.SemaphoreType.DMA,
    ],
    compiler_params=_sc_params,
)
def _gather_k(h_hbm, idx_hbm, out_hbm, idx_v, rows_v, sem):
    wid = lax.axis_index("s") * NC + lax.axis_index("c")
    pltpu.sync_copy(idx_hbm.at[wid], idx_v)
    G = 10

    def body(g, carry):
        j0 = g * G
        descs = [
            pltpu.async_copy(h_hbm.at[idx_v.at[j0 + j]], rows_v.at[j0 + j], sem)
            for j in range(G)
        ]
        for d in descs:
            d.wait()
        return carry

    lax.fori_loop(0, NCH // G, body, 0)
    pltpu.sync_copy(rows_v, out_hbm.at[wid])


@functools.partial(
    pl.kernel,
    out_type=jax.ShapeDtypeStruct((NC, NS, ROWS_PER_TILE, C), jnp.float32),
    mesh=_mesh,
    scratch_types=[
        pltpu.VMEM((NCH, CHUNK), jnp.int32),
        pltpu.VMEM((NCH, CHUNK, C), jnp.float32),
        pltpu.VMEM_SHARED((N_NODES, C), jnp.float32),
        pltpu.SemaphoreType.DMA,
    ],
    compiler_params=_sc_params,
)
def _scatter_k(m_hbm, dst_hbm, zero_hbm, out_hbm, idx_v, m_v, acc, sem):
    sid = lax.axis_index("s")
    cid = lax.axis_index("c")
    wid = sid * NC + cid
    pltpu.sync_copy(zero_hbm, acc.at[pl.ds(sid * ROWS_PER_TILE, ROWS_PER_TILE)])
    plsc.subcore_barrier()
    pltpu.sync_copy(dst_hbm.at[wid], idx_v)
    pltpu.sync_copy(m_hbm.at[wid], m_v)

    def body(j, carry):
        pltpu.sync_copy(m_v.at[j], acc.at[idx_v.at[j]], add=True)
        return carry

    lax.fori_loop(0, NCH, body, 0)
    plsc.subcore_barrier()
    pltpu.sync_copy(
        acc.at[pl.ds(sid * ROWS_PER_TILE, ROWS_PER_TILE)],
        out_hbm.at[cid, sid],
    )


def _messages_body(hs_ref, e_ref, ew_ref, eb_ref, rep_ref, sum_ref, m_ref):
    w = (
        jnp.dot(e_ref[...], ew_ref[...], preferred_element_type=jnp.float32,
                precision=lax.Precision.HIGHEST)
        + eb_ref[...]
    )
    hr = jnp.dot(hs_ref[...], rep_ref[...], preferred_element_type=jnp.float32,
                 precision=lax.Precision.HIGHEST)
    u = hr * w
    m_ref[...] = jnp.dot(u, sum_ref[...], preferred_element_type=jnp.float32,
                         precision=lax.Precision.HIGHEST)


def _messages(h_src, e, edge_w, edge_b):
    BLK = 2000
    grid = (N_EDGES // BLK,)
    full = lambda shape: pl.BlockSpec(shape, lambda i: (0, 0))
    return pl.pallas_call(
        _messages_body,
        grid=grid,
        in_specs=[
            pl.BlockSpec((BLK, C), lambda i: (i, 0)),
            pl.BlockSpec((BLK, C), lambda i: (i, 0)),
            full((C, C * C)),
            full((1, C * C)),
            full((C, C * C)),
            full((C * C, C)),
        ],
        out_specs=pl.BlockSpec((BLK, C), lambda i: (i, 0)),
        out_shape=jax.ShapeDtypeStruct((N_EDGES, C), jnp.float32),
    )(h_src, e, edge_w, edge_b.reshape(1, C * C), jnp.asarray(_REP), jnp.asarray(_SUM))


def _combine_body(h_ref, root_ref, bias_ref, p_ref, out_ref):
    out_ref[...] = (
        jnp.dot(h_ref[...], root_ref[...], preferred_element_type=jnp.float32,
                precision=lax.Precision.HIGHEST)
        + bias_ref[...]
        + p_ref[0]
        + p_ref[1]
    )


def _combine(h, root, bias, partials):
    return pl.pallas_call(
        _combine_body,
        out_shape=jax.ShapeDtypeStruct((N_NODES, C), jnp.float32),
    )(h, root, bias.reshape(1, C), partials)


def kernel(h, e, edge_index, edge_w, edge_b, root, bias):
    src = edge_index[0].reshape(NW, NCH, CHUNK)
    dst = edge_index[1].reshape(NW, NCH, CHUNK)
    h_src = _gather_k(h, src).reshape(N_EDGES, C)
    m = _messages(h_src, e, edge_w, edge_b)
    zeros = jnp.zeros((ROWS_PER_TILE, C), jnp.float32)
    partials = _scatter_k(m.reshape(NW, NCH, CHUNK, C), dst, zeros)
    return _combine(h, root, bias, partials.reshape(NC, N_NODES, C))


# speedup vs baseline: 3.6705x; 2.3776x over previous
import functools

import jax
import jax.numpy as jnp
import numpy as np
from jax import lax
from jax.experimental import pallas as pl
from jax.experimental.pallas import tpu as pltpu
from jax.experimental.pallas import tpu_sc as plsc

N_NODES = 10000
N_EDGES = 160000
C = 16

NC = 2
NS = 16
NW = NC * NS
CHUNK = 100
EPW = N_EDGES // NW
NCH = EPW // CHUNK
EPT = N_EDGES // NS
NCHS = EPT // CHUNK
HALF = NCHS // 2
ROWS_PER_TILE = N_NODES // NS

_REP = np.repeat(np.eye(C, dtype=np.float32), C, axis=1)
_SUM = np.tile(np.eye(C, dtype=np.float32), (C, 1))

_mesh = plsc.VectorSubcoreMesh(core_axis_name="c", subcore_axis_name="s")
_sc_params = pltpu.CompilerParams(use_tc_tiling_on_sc=False)


@functools.partial(
    pl.kernel,
    out_type=jax.ShapeDtypeStruct((NW, NCH, CHUNK, C), jnp.float32),
    mesh=_mesh,
    scratch_types=[
        pltpu.VMEM((NCH, CHUNK), jnp.int32),
        pltpu.VMEM((NCH, CHUNK, C), jnp.float32),
        pltpu.SemaphoreType.DMA,
    ],
    compiler_params=_sc_params,
)
def _gather_k(h_hbm, idx_hbm, out_hbm, idx_v, rows_v, sem):
    wid = lax.axis_index("s") * NC + lax.axis_index("c")
    pltpu.sync_copy(idx_hbm.at[wid], idx_v)
    G = 10

    def body(g, carry):
        j0 = g * G
        descs = [
            pltpu.async_copy(h_hbm.at[idx_v.at[j0 + j]], rows_v.at[j0 + j], sem)
            for j in range(G)
        ]
        for d in descs:
            d.wait()
        return carry

    lax.fori_loop(0, NCH // G, body, 0)
    pltpu.sync_copy(rows_v, out_hbm.at[wid])


@functools.partial(
    pl.kernel,
    out_type=jax.ShapeDtypeStruct((NS, ROWS_PER_TILE, C), jnp.float32),
    mesh=_mesh,
    scratch_types=[
        pltpu.VMEM((HALF, CHUNK), jnp.int32),
        pltpu.VMEM((HALF, CHUNK, C), jnp.float32),
        pltpu.VMEM_SHARED((N_NODES, C), jnp.float32),
        pltpu.SemaphoreType.DMA,
    ],
    compiler_params=_sc_params,
)
def _scatter_k(m_hbm, dst_hbm, base_hbm, out_hbm, idx_v, m_v, acc, sem):
    sid = lax.axis_index("s")
    cid = lax.axis_index("c")

    @pl.when(cid == 0)
    def _():
        pltpu.sync_copy(base_hbm.at[sid], acc.at[pl.ds(sid * ROWS_PER_TILE, ROWS_PER_TILE)])
        plsc.subcore_barrier()

        def half(p):
            pltpu.sync_copy(dst_hbm.at[sid, pl.ds(p * HALF, HALF)], idx_v)
            pltpu.sync_copy(m_hbm.at[sid, pl.ds(p * HALF, HALF)], m_v)

            def body(j, carry):
                pltpu.sync_copy(m_v.at[j], acc.at[idx_v.at[j]], add=True)
                return carry

            lax.fori_loop(0, HALF, body, 0)

        half(0)
        half(1)
        plsc.subcore_barrier()
        pltpu.sync_copy(
            acc.at[pl.ds(sid * ROWS_PER_TILE, ROWS_PER_TILE)],
            out_hbm.at[sid],
        )


def _messages_body(hs_ref, e_ref, ew_ref, eb_ref, h_ref, root_ref, bias_ref,
                   rep_ref, sum_ref, m_ref, ob_ref):
    w = jnp.dot(e_ref[...], ew_ref[...], preferred_element_type=jnp.float32) + eb_ref[...]
    hr = jnp.dot(hs_ref[...], rep_ref[...], preferred_element_type=jnp.float32)
    m_ref[...] = jnp.dot(hr * w, sum_ref[...], preferred_element_type=jnp.float32)

    @pl.when(pl.program_id(0) == 0)
    def _():
        ob_ref[...] = (
            jnp.dot(h_ref[...], root_ref[...], preferred_element_type=jnp.float32)
            + bias_ref[...]
        )


def _messages(h_src, e, edge_w, edge_b, h, root, bias):
    BLK = 2000
    grid = (N_EDGES // BLK,)
    full = lambda shape: pl.BlockSpec(shape, lambda i: tuple(0 for _ in shape))
    return pl.pallas_call(
        _messages_body,
        grid=grid,
        in_specs=[
            pl.BlockSpec((BLK, C), lambda i: (i, 0)),
            pl.BlockSpec((BLK, C), lambda i: (i, 0)),
            full((C, C * C)),
            full((1, C * C)),
            full((N_NODES, C)),
            full((C, C)),
            full((1, C)),
            full((C, C * C)),
            full((C * C, C)),
        ],
        out_specs=[
            pl.BlockSpec((BLK, C), lambda i: (i, 0)),
            full((N_NODES, C)),
        ],
        out_shape=[
            jax.ShapeDtypeStruct((N_EDGES, C), jnp.float32),
            jax.ShapeDtypeStruct((N_NODES, C), jnp.float32),
        ],
    )(h_src, e, edge_w, edge_b.reshape(1, C * C), h, root, bias.reshape(1, C),
      jnp.asarray(_REP), jnp.asarray(_SUM))


def kernel(h, e, edge_index, edge_w, edge_b, root, bias):
    src = edge_index[0].reshape(NW, NCH, CHUNK)
    dst = edge_index[1].reshape(NS, NCHS, CHUNK)
    h_src = _gather_k(h, src).reshape(N_EDGES, C)
    m, out_base = _messages(h_src, e, edge_w, edge_b, h, root, bias)
    out = _scatter_k(
        m.reshape(NS, NCHS, CHUNK, C),
        dst,
        out_base.reshape(NS, ROWS_PER_TILE, C),
    )
    return out.reshape(N_NODES, C)


# speedup vs baseline: 3.6969x; 1.0072x over previous
import functools

import jax
import jax.numpy as jnp
import numpy as np
from jax import lax
from jax.experimental import pallas as pl
from jax.experimental.pallas import tpu as pltpu
from jax.experimental.pallas import tpu_sc as plsc

N_NODES = 10000
N_EDGES = 160000
C = 16

NC = 2
NS = 16
NW = NC * NS
CHUNK = 100
EPW = N_EDGES // NW
NCH = EPW // CHUNK
EPT = N_EDGES // NS
NCHS = EPT // CHUNK
HALF_E = EPT // 2
HALF = NCHS // 2
STRIPE = 640
LAST_STRIPE = N_NODES - STRIPE * (NS - 1)

_REP = np.repeat(np.eye(C, dtype=np.float32), C, axis=1)
_SUM = np.tile(np.eye(C, dtype=np.float32), (C, 1))

_mesh = plsc.VectorSubcoreMesh(core_axis_name="c", subcore_axis_name="s")
_sc_params = pltpu.CompilerParams(use_tc_tiling_on_sc=False)


@functools.partial(
    pl.kernel,
    out_type=jax.ShapeDtypeStruct((N_EDGES, C), jnp.float32),
    mesh=_mesh,
    scratch_types=[
        pltpu.VMEM((NCH, CHUNK), jnp.int32),
        pltpu.VMEM((EPW, C), jnp.float32),
        pltpu.SemaphoreType.DMA,
    ],
    compiler_params=_sc_params,
)
def _gather_k(h_hbm, idx_hbm, out_hbm, idx_v, rows_v, sem):
    wid = lax.axis_index("s") * NC + lax.axis_index("c")
    pltpu.sync_copy(idx_hbm.at[wid], idx_v)
    G = 10

    def body(g, carry):
        j0 = g * G
        descs = [
            pltpu.async_copy(
                h_hbm.at[idx_v.at[j0 + j]],
                rows_v.at[pl.ds((j0 + j) * CHUNK, CHUNK)],
                sem,
            )
            for j in range(G)
        ]
        for d in descs:
            d.wait()
        return carry

    lax.fori_loop(0, NCH // G, body, 0)
    pltpu.sync_copy(rows_v, out_hbm.at[pl.ds(wid * EPW, EPW)])


@functools.partial(
    pl.kernel,
    out_type=jax.ShapeDtypeStruct((N_NODES, C), jnp.float32),
    mesh=_mesh,
    scratch_types=[
        pltpu.VMEM((HALF, CHUNK), jnp.int32),
        pltpu.VMEM((HALF_E, C), jnp.float32),
        pltpu.VMEM_SHARED((N_NODES, C), jnp.float32),
        pltpu.SemaphoreType.DMA,
    ],
    compiler_params=_sc_params,
)
def _scatter_k(m_hbm, dst_hbm, base_hbm, out_hbm, idx_v, m_v, acc, sem):
    sid = lax.axis_index("s")
    cid = lax.axis_index("c")

    @pl.when(cid == 0)
    def _():
        row0 = sid * STRIPE

        @pl.when(sid < NS - 1)
        def _():
            pltpu.sync_copy(base_hbm.at[pl.ds(row0, STRIPE)],
                            acc.at[pl.ds(row0, STRIPE)])

        @pl.when(sid == NS - 1)
        def _():
            pltpu.sync_copy(base_hbm.at[pl.ds(row0, LAST_STRIPE)],
                            acc.at[pl.ds(row0, LAST_STRIPE)])

        plsc.subcore_barrier()

        def half(p):
            pltpu.sync_copy(dst_hbm.at[sid, pl.ds(p * HALF, HALF)], idx_v)
            pltpu.sync_copy(
                m_hbm.at[pl.ds(sid * EPT + p * HALF_E, HALF_E)], m_v
            )

            def body(j, carry):
                pltpu.sync_copy(
                    m_v.at[pl.ds(j * CHUNK, CHUNK)], acc.at[idx_v.at[j]], add=True
                )
                return carry

            lax.fori_loop(0, HALF, body, 0)

        half(0)
        half(1)
        plsc.subcore_barrier()

        @pl.when(sid < NS - 1)
        def _():
            pltpu.sync_copy(acc.at[pl.ds(row0, STRIPE)],
                            out_hbm.at[pl.ds(row0, STRIPE)])

        @pl.when(sid == NS - 1)
        def _():
            pltpu.sync_copy(acc.at[pl.ds(row0, LAST_STRIPE)],
                            out_hbm.at[pl.ds(row0, LAST_STRIPE)])


def _messages_body(hs_ref, e_ref, ew_ref, eb_ref, h_ref, root_ref, bias_ref,
                   rep_ref, sum_ref, m_ref, ob_ref):
    w = jnp.dot(e_ref[...], ew_ref[...], preferred_element_type=jnp.float32) + eb_ref[...]
    hr = jnp.dot(hs_ref[...], rep_ref[...], preferred_element_type=jnp.float32)
    m_ref[...] = jnp.dot(hr * w, sum_ref[...], preferred_element_type=jnp.float32)

    @pl.when(pl.program_id(0) == 0)
    def _():
        ob_ref[...] = (
            jnp.dot(h_ref[...], root_ref[...], preferred_element_type=jnp.float32)
            + bias_ref[...]
        )


def _messages(h_src, e, edge_w, edge_b, h, root, bias):
    BLK = 2000
    grid = (N_EDGES // BLK,)
    full = lambda shape: pl.BlockSpec(shape, lambda i: tuple(0 for _ in shape))
    return pl.pallas_call(
        _messages_body,
        grid=grid,
        in_specs=[
            pl.BlockSpec((BLK, C), lambda i: (i, 0)),
            pl.BlockSpec((BLK, C), lambda i: (i, 0)),
            full((C, C * C)),
            full((1, C * C)),
            full((N_NODES, C)),
            full((C, C)),
            full((1, C)),
            full((C, C * C)),
            full((C * C, C)),
        ],
        out_specs=[
            pl.BlockSpec((BLK, C), lambda i: (i, 0)),
            full((N_NODES, C)),
        ],
        out_shape=[
            jax.ShapeDtypeStruct((N_EDGES, C), jnp.float32),
            jax.ShapeDtypeStruct((N_NODES, C), jnp.float32),
        ],
    )(h_src, e, edge_w, edge_b.reshape(1, C * C), h, root, bias.reshape(1, C),
      jnp.asarray(_REP), jnp.asarray(_SUM))


def kernel(h, e, edge_index, edge_w, edge_b, root, bias):
    src = edge_index[0].reshape(NW, NCH, CHUNK)
    dst = edge_index[1].reshape(NS, NCHS, CHUNK)
    h_src = _gather_k(h, src)
    m, out_base = _messages(h_src, e, edge_w, edge_b, h, root, bias)
    return _scatter_k(m, dst, out_base)


# speedup vs baseline: 5.6198x; 1.5201x over previous
import functools

import jax
import jax.numpy as jnp
import numpy as np
from jax import lax
from jax.experimental import pallas as pl
from jax.experimental.pallas import tpu as pltpu
from jax.experimental.pallas import tpu_sc as plsc

N_NODES = 10000
N_EDGES = 160000
C = 16

NC = 2
NS = 16
NW = NC * NS
CHUNK = 100
EPW = N_EDGES // NW
NCH = EPW // CHUNK
EPT = N_EDGES // NS
NCHS = EPT // CHUNK
HALF_E = EPT // 2
HALF = NCHS // 2
STRIPE = 640
LAST_STRIPE = N_NODES - STRIPE * (NS - 1)

PK = 8
EROWS = N_EDGES // PK
NROWS = N_NODES // PK
_REP = np.repeat(np.eye(C, dtype=np.float32), C, axis=1)
_SUM = np.tile(np.eye(C, dtype=np.float32), (C, 1))
_REP8 = np.kron(np.eye(PK, dtype=np.float32), _REP)
_SUM8 = np.kron(np.eye(PK, dtype=np.float32), _SUM)

_mesh = plsc.VectorSubcoreMesh(core_axis_name="c", subcore_axis_name="s")
_sc_params = pltpu.CompilerParams(use_tc_tiling_on_sc=False)


@functools.partial(
    pl.kernel,
    out_type=jax.ShapeDtypeStruct((N_EDGES, C), jnp.float32),
    mesh=_mesh,
    scratch_types=[
        pltpu.VMEM((NCH, CHUNK), jnp.int32),
        pltpu.VMEM((EPW, C), jnp.float32),
        pltpu.SemaphoreType.DMA,
    ],
    compiler_params=_sc_params,
)
def _gather_k(h_hbm, idx_hbm, out_hbm, idx_v, rows_v, sem):
    wid = lax.axis_index("s") * NC + lax.axis_index("c")
    pltpu.sync_copy(idx_hbm.at[wid], idx_v)
    G = 10

    def body(g, carry):
        j0 = g * G
        descs = [
            pltpu.async_copy(
                h_hbm.at[idx_v.at[j0 + j]],
                rows_v.at[pl.ds((j0 + j) * CHUNK, CHUNK)],
                sem,
            )
            for j in range(G)
        ]
        for d in descs:
            d.wait()
        return carry

    lax.fori_loop(0, NCH // G, body, 0)
    pltpu.sync_copy(rows_v, out_hbm.at[pl.ds(wid * EPW, EPW)])


@functools.partial(
    pl.kernel,
    out_type=jax.ShapeDtypeStruct((N_NODES, C), jnp.float32),
    mesh=_mesh,
    scratch_types=[
        pltpu.VMEM((HALF, CHUNK), jnp.int32),
        pltpu.VMEM((HALF_E, C), jnp.float32),
        pltpu.VMEM_SHARED((N_NODES, C), jnp.float32),
        pltpu.SemaphoreType.DMA,
    ],
    compiler_params=_sc_params,
)
def _scatter_k(m_hbm, dst_hbm, base_hbm, out_hbm, idx_v, m_v, acc, sem):
    sid = lax.axis_index("s")
    cid = lax.axis_index("c")

    @pl.when(cid == 0)
    def _():
        row0 = sid * STRIPE

        @pl.when(sid < NS - 1)
        def _():
            pltpu.sync_copy(base_hbm.at[pl.ds(row0, STRIPE)],
                            acc.at[pl.ds(row0, STRIPE)])

        @pl.when(sid == NS - 1)
        def _():
            pltpu.sync_copy(base_hbm.at[pl.ds(row0, LAST_STRIPE)],
                            acc.at[pl.ds(row0, LAST_STRIPE)])

        plsc.subcore_barrier()

        def half(p):
            pltpu.sync_copy(dst_hbm.at[sid, pl.ds(p * HALF, HALF)], idx_v)
            pltpu.sync_copy(
                m_hbm.at[pl.ds(sid * EPT + p * HALF_E, HALF_E)], m_v
            )

            def body(j, carry):
                pltpu.sync_copy(
                    m_v.at[pl.ds(j * CHUNK, CHUNK)], acc.at[idx_v.at[j]], add=True
                )
                return carry

            lax.fori_loop(0, HALF, body, 0)

        half(0)
        half(1)
        plsc.subcore_barrier()

        @pl.when(sid < NS - 1)
        def _():
            pltpu.sync_copy(acc.at[pl.ds(row0, STRIPE)],
                            out_hbm.at[pl.ds(row0, STRIPE)])

        @pl.when(sid == NS - 1)
        def _():
            pltpu.sync_copy(acc.at[pl.ds(row0, LAST_STRIPE)],
                            out_hbm.at[pl.ds(row0, LAST_STRIPE)])


def _messages_body(hsp_ref, ep_ref, ew8_ref, b8_ref, hp_ref, root8_ref,
                   bias8_ref, rep8_ref, sum8_ref, m_ref, ob_ref):
    w = jnp.dot(ep_ref[...], ew8_ref[...], preferred_element_type=jnp.float32) + b8_ref[...]
    hr = jnp.dot(hsp_ref[...], rep8_ref[...], preferred_element_type=jnp.float32)
    m_ref[...] = jnp.dot(hr * w, sum8_ref[...], preferred_element_type=jnp.float32)

    @pl.when(pl.program_id(0) == 0)
    def _():
        ob_ref[...] = (
            jnp.dot(hp_ref[...], root8_ref[...], preferred_element_type=jnp.float32)
            + bias8_ref[...]
        )


def _messages(hs_pack, e_pack, edge_w, edge_b, h_pack, root, bias):
    BLKP = 400
    grid = (EROWS // BLKP,)
    full = lambda shape: pl.BlockSpec(shape, lambda i: tuple(0 for _ in shape))
    ew8 = jnp.kron(jnp.eye(PK, dtype=jnp.float32), edge_w)
    b8 = jnp.tile(edge_b, PK).reshape(1, PK * C * C)
    root8 = jnp.kron(jnp.eye(PK, dtype=jnp.float32), root)
    bias8 = jnp.tile(bias, PK).reshape(1, PK * C)
    return pl.pallas_call(
        _messages_body,
        grid=grid,
        in_specs=[
            pl.BlockSpec((BLKP, PK * C), lambda i: (i, 0)),
            pl.BlockSpec((BLKP, PK * C), lambda i: (i, 0)),
            full((PK * C, PK * C * C)),
            full((1, PK * C * C)),
            full((NROWS, PK * C)),
            full((PK * C, PK * C)),
            full((1, PK * C)),
            full((PK * C, PK * C * C)),
            full((PK * C * C, PK * C)),
        ],
        out_specs=[
            pl.BlockSpec((BLKP, PK * C), lambda i: (i, 0)),
            full((NROWS, PK * C)),
        ],
        out_shape=[
            jax.ShapeDtypeStruct((EROWS, PK * C), jnp.float32),
            jax.ShapeDtypeStruct((NROWS, PK * C), jnp.float32),
        ],
    )(hs_pack, e_pack, ew8, b8, h_pack, root8, bias8,
      jnp.asarray(_REP8), jnp.asarray(_SUM8))


def kernel(h, e, edge_index, edge_w, edge_b, root, bias):
    src = edge_index[0].reshape(NW, NCH, CHUNK)
    dst = edge_index[1].reshape(NS, NCHS, CHUNK)
    h_src = _gather_k(h, src)
    m_pack, ob_pack = _messages(
        h_src.reshape(EROWS, PK * C),
        e.reshape(EROWS, PK * C),
        edge_w, edge_b,
        h.reshape(NROWS, PK * C),
        root, bias,
    )
    return _scatter_k(
        m_pack.reshape(N_EDGES, C), dst, ob_pack.reshape(N_NODES, C)
    )


# speedup vs baseline: 5.8751x; 1.0454x over previous
import functools

import jax
import jax.numpy as jnp
import numpy as np
from jax import lax
from jax.experimental import pallas as pl
from jax.experimental.pallas import tpu as pltpu
from jax.experimental.pallas import tpu_sc as plsc

N_NODES = 10000
N_EDGES = 160000
C = 16

NC = 2
NS = 16
NW = NC * NS
CHUNK = 100
EPW = N_EDGES // NW
NCH = EPW // CHUNK
EPT = N_EDGES // NS
NCHS = EPT // CHUNK
HALF_E = EPT // 2
HALF = NCHS // 2
STRIPE = 640
LAST_STRIPE = N_NODES - STRIPE * (NS - 1)

PK = 8
EROWS = N_EDGES // PK
NROWS = N_NODES // PK
_REP = np.repeat(np.eye(C, dtype=np.float32), C, axis=1)
_SUM = np.tile(np.eye(C, dtype=np.float32), (C, 1))
_REP8 = np.kron(np.eye(PK, dtype=np.float32), _REP)
_SUM8 = np.kron(np.eye(PK, dtype=np.float32), _SUM)

_mesh = plsc.VectorSubcoreMesh(core_axis_name="c", subcore_axis_name="s")
_sc_params = pltpu.CompilerParams(use_tc_tiling_on_sc=False)


@functools.partial(
    pl.kernel,
    out_type=jax.ShapeDtypeStruct((N_EDGES, C), jnp.float32),
    mesh=_mesh,
    scratch_types=[
        pltpu.VMEM((NCH, CHUNK), jnp.int32),
        pltpu.VMEM((EPW, C), jnp.float32),
        pltpu.SemaphoreType.DMA,
    ],
    compiler_params=_sc_params,
)
def _gather_k(h_hbm, idx_hbm, out_hbm, idx_v, rows_v, sem):
    wid = lax.axis_index("s") * NC + lax.axis_index("c")
    pltpu.sync_copy(idx_hbm.at[wid], idx_v)
    G = 10

    def body(g, carry):
        j0 = g * G
        descs = [
            pltpu.async_copy(
                h_hbm.at[idx_v.at[j0 + j]],
                rows_v.at[pl.ds((j0 + j) * CHUNK, CHUNK)],
                sem,
            )
            for j in range(G)
        ]
        for d in descs:
            d.wait()
        return carry

    lax.fori_loop(0, NCH // G, body, 0)
    pltpu.sync_copy(rows_v, out_hbm.at[pl.ds(wid * EPW, EPW)])


@functools.partial(
    pl.kernel,
    out_type=jax.ShapeDtypeStruct((N_NODES, C), jnp.float32),
    mesh=_mesh,
    scratch_types=[
        pltpu.VMEM((HALF, CHUNK), jnp.int32),
        pltpu.VMEM((HALF_E, C), jnp.float32),
        pltpu.VMEM_SHARED((N_NODES, C), jnp.float32),
        pltpu.SemaphoreType.DMA,
    ],
    compiler_params=_sc_params,
)
def _scatter_k(m_hbm, dst_hbm, base_hbm, out_hbm, idx_v, m_v, acc, sem):
    sid = lax.axis_index("s")
    cid = lax.axis_index("c")

    @pl.when(cid == 0)
    def _():
        row0 = sid * STRIPE

        @pl.when(sid < NS - 1)
        def _():
            pltpu.sync_copy(base_hbm.at[pl.ds(row0, STRIPE)],
                            acc.at[pl.ds(row0, STRIPE)])

        @pl.when(sid == NS - 1)
        def _():
            pltpu.sync_copy(base_hbm.at[pl.ds(row0, LAST_STRIPE)],
                            acc.at[pl.ds(row0, LAST_STRIPE)])

        plsc.subcore_barrier()

        def half(p):
            pltpu.sync_copy(dst_hbm.at[sid, pl.ds(p * HALF, HALF)], idx_v)
            pltpu.sync_copy(
                m_hbm.at[pl.ds(sid * EPT + p * HALF_E, HALF_E)], m_v
            )

            def body(j, carry):
                pltpu.sync_copy(
                    m_v.at[pl.ds(j * CHUNK, CHUNK)], acc.at[idx_v.at[j]], add=True
                )
                return carry

            lax.fori_loop(0, HALF, body, 0)

        half(0)
        half(1)
        plsc.subcore_barrier()

        @pl.when(sid < NS - 1)
        def _():
            pltpu.sync_copy(acc.at[pl.ds(row0, STRIPE)],
                            out_hbm.at[pl.ds(row0, STRIPE)])

        @pl.when(sid == NS - 1)
        def _():
            pltpu.sync_copy(acc.at[pl.ds(row0, LAST_STRIPE)],
                            out_hbm.at[pl.ds(row0, LAST_STRIPE)])


def _messages_body(hsp_ref, ep_ref, ew8_ref, b8_ref, hp_ref, root8_ref,
                   bias8_ref, rep8_ref, sum8_ref, m_ref, ob_ref):
    w = jnp.dot(ep_ref[...], ew8_ref[...], preferred_element_type=jnp.float32) + b8_ref[...]
    hr = jnp.dot(hsp_ref[...], rep8_ref[...], preferred_element_type=jnp.float32)
    m_ref[...] = jnp.dot(hr * w, sum8_ref[...], preferred_element_type=jnp.float32)

    @pl.when(pl.program_id(0) == 0)
    def _():
        ob_ref[...] = (
            jnp.dot(hp_ref[...], root8_ref[...], preferred_element_type=jnp.float32)
            + bias8_ref[...]
        )


def _messages(hs_pack, e_pack, edge_w, edge_b, h_pack, root, bias):
    BLKP = 2000
    grid = (EROWS // BLKP,)
    full = lambda shape: pl.BlockSpec(shape, lambda i: tuple(0 for _ in shape))
    ew8 = jnp.kron(jnp.eye(PK, dtype=jnp.float32), edge_w)
    b8 = jnp.tile(edge_b, PK).reshape(1, PK * C * C)
    root8 = jnp.kron(jnp.eye(PK, dtype=jnp.float32), root)
    bias8 = jnp.tile(bias, PK).reshape(1, PK * C)
    return pl.pallas_call(
        _messages_body,
        grid=grid,
        in_specs=[
            pl.BlockSpec((BLKP, PK * C), lambda i: (i, 0)),
            pl.BlockSpec((BLKP, PK * C), lambda i: (i, 0)),
            full((PK * C, PK * C * C)),
            full((1, PK * C * C)),
            full((NROWS, PK * C)),
            full((PK * C, PK * C)),
            full((1, PK * C)),
            full((PK * C, PK * C * C)),
            full((PK * C * C, PK * C)),
        ],
        out_specs=[
            pl.BlockSpec((BLKP, PK * C), lambda i: (i, 0)),
            full((NROWS, PK * C)),
        ],
        out_shape=[
            jax.ShapeDtypeStruct((EROWS, PK * C), jnp.float32),
            jax.ShapeDtypeStruct((NROWS, PK * C), jnp.float32),
        ],
    )(hs_pack, e_pack, ew8, b8, h_pack, root8, bias8,
      jnp.asarray(_REP8), jnp.asarray(_SUM8))


def kernel(h, e, edge_index, edge_w, edge_b, root, bias):
    src = edge_index[0].reshape(NW, NCH, CHUNK)
    dst = edge_index[1].reshape(NS, NCHS, CHUNK)
    h_src = _gather_k(h, src)
    m_pack, ob_pack = _messages(
        h_src.reshape(EROWS, PK * C),
        e.reshape(EROWS, PK * C),
        edge_w, edge_b,
        h.reshape(NROWS, PK * C),
        root, bias,
    )
    return _scatter_k(
        m_pack.reshape(N_EDGES, C), dst, ob_pack.reshape(N_NODES, C)
    )


# speedup vs baseline: 6.0485x; 1.0295x over previous
import functools

import jax
import jax.numpy as jnp
import numpy as np
from jax import lax
from jax.experimental import pallas as pl
from jax.experimental.pallas import tpu as pltpu
from jax.experimental.pallas import tpu_sc as plsc

N_NODES = 10000
N_EDGES = 160000
C = 16

NC = 2
NS = 16
NW = NC * NS
CHUNK = 100
EPW = N_EDGES // NW
NCH = EPW // CHUNK
EPT = N_EDGES // NS
NCHS = EPT // CHUNK
HALF_E = EPT // 2
HALF = NCHS // 2
STRIPE = 640
LAST_STRIPE = N_NODES - STRIPE * (NS - 1)

PK = 8
EROWS = N_EDGES // PK
NROWS = N_NODES // PK
_REP = np.repeat(np.eye(C, dtype=np.float32), C, axis=1)
_SUM = np.tile(np.eye(C, dtype=np.float32), (C, 1))
_REP8 = np.kron(np.eye(PK, dtype=np.float32), _REP)
_SUM8 = np.kron(np.eye(PK, dtype=np.float32), _SUM)

_mesh = plsc.VectorSubcoreMesh(core_axis_name="c", subcore_axis_name="s")
_sc_params = pltpu.CompilerParams(use_tc_tiling_on_sc=False)


@functools.partial(
    pl.kernel,
    out_type=jax.ShapeDtypeStruct((N_EDGES, C), jnp.float32),
    mesh=_mesh,
    scratch_types=[
        pltpu.VMEM((NCH, CHUNK), jnp.int32),
        pltpu.VMEM((EPW, C), jnp.float32),
        pltpu.SemaphoreType.DMA,
    ],
    compiler_params=_sc_params,
)
def _gather_k(h_hbm, idx_hbm, out_hbm, idx_v, rows_v, sem):
    wid = lax.axis_index("s") * NC + lax.axis_index("c")
    pltpu.sync_copy(idx_hbm.at[wid], idx_v)
    G = 10

    def body(g, carry):
        j0 = g * G
        descs = [
            pltpu.async_copy(
                h_hbm.at[idx_v.at[j0 + j]],
                rows_v.at[pl.ds((j0 + j) * CHUNK, CHUNK)],
                sem,
            )
            for j in range(G)
        ]
        for d in descs:
            d.wait()
        return carry

    lax.fori_loop(0, NCH // G, body, 0)
    pltpu.sync_copy(rows_v, out_hbm.at[pl.ds(wid * EPW, EPW)])


@functools.partial(
    pl.kernel,
    out_type=jax.ShapeDtypeStruct((N_NODES, C), jnp.float32),
    mesh=_mesh,
    scratch_types=[
        pltpu.VMEM((HALF, CHUNK), jnp.int32),
        pltpu.VMEM((HALF_E, C), jnp.float32),
        pltpu.VMEM_SHARED((N_NODES, C), jnp.float32),
        pltpu.SemaphoreType.DMA,
    ],
    compiler_params=_sc_params,
)
def _scatter_k(m_hbm, dst_hbm, base_hbm, out_hbm, idx_v, m_v, acc, sem):
    sid = lax.axis_index("s")
    cid = lax.axis_index("c")

    @pl.when(cid == 0)
    def _():
        row0 = sid * STRIPE

        @pl.when(sid < NS - 1)
        def _():
            pltpu.sync_copy(base_hbm.at[pl.ds(row0, STRIPE)],
                            acc.at[pl.ds(row0, STRIPE)])

        @pl.when(sid == NS - 1)
        def _():
            pltpu.sync_copy(base_hbm.at[pl.ds(row0, LAST_STRIPE)],
                            acc.at[pl.ds(row0, LAST_STRIPE)])

        plsc.subcore_barrier()

        def half(p):
            pltpu.sync_copy(dst_hbm.at[sid, pl.ds(p * HALF, HALF)], idx_v)
            pltpu.sync_copy(
                m_hbm.at[pl.ds(sid * EPT + p * HALF_E, HALF_E)], m_v
            )

            G = 10

            def body(g, carry):
                j0 = g * G
                descs = [
                    pltpu.async_copy(
                        m_v.at[pl.ds((j0 + j) * CHUNK, CHUNK)],
                        acc.at[idx_v.at[j0 + j]],
                        sem,
                        add=True,
                    )
                    for j in range(G)
                ]
                for d in descs:
                    d.wait()
                return carry

            lax.fori_loop(0, HALF // G, body, 0)

        half(0)
        half(1)
        plsc.subcore_barrier()

        @pl.when(sid < NS - 1)
        def _():
            pltpu.sync_copy(acc.at[pl.ds(row0, STRIPE)],
                            out_hbm.at[pl.ds(row0, STRIPE)])

        @pl.when(sid == NS - 1)
        def _():
            pltpu.sync_copy(acc.at[pl.ds(row0, LAST_STRIPE)],
                            out_hbm.at[pl.ds(row0, LAST_STRIPE)])


def _messages_body(hsp_ref, ep_ref, ew8_ref, b8_ref, hp_ref, root8_ref,
                   bias8_ref, rep8_ref, sum8_ref, m_ref, ob_ref):
    w = jnp.dot(ep_ref[...], ew8_ref[...], preferred_element_type=jnp.float32) + b8_ref[...]
    hr = jnp.dot(hsp_ref[...], rep8_ref[...], preferred_element_type=jnp.float32)
    m_ref[...] = jnp.dot(hr * w, sum8_ref[...], preferred_element_type=jnp.float32)

    @pl.when(pl.program_id(0) == 0)
    def _():
        ob_ref[...] = (
            jnp.dot(hp_ref[...], root8_ref[...], preferred_element_type=jnp.float32)
            + bias8_ref[...]
        )


def _messages(hs_pack, e_pack, edge_w, edge_b, h_pack, root, bias):
    BLKP = 2000
    grid = (EROWS // BLKP,)
    full = lambda shape: pl.BlockSpec(shape, lambda i: tuple(0 for _ in shape))
    ew8 = jnp.kron(jnp.eye(PK, dtype=jnp.float32), edge_w)
    b8 = jnp.tile(edge_b, PK).reshape(1, PK * C * C)
    root8 = jnp.kron(jnp.eye(PK, dtype=jnp.float32), root)
    bias8 = jnp.tile(bias, PK).reshape(1, PK * C)
    return pl.pallas_call(
        _messages_body,
        grid=grid,
        in_specs=[
            pl.BlockSpec((BLKP, PK * C), lambda i: (i, 0)),
            pl.BlockSpec((BLKP, PK * C), lambda i: (i, 0)),
            full((PK * C, PK * C * C)),
            full((1, PK * C * C)),
            full((NROWS, PK * C)),
            full((PK * C, PK * C)),
            full((1, PK * C)),
            full((PK * C, PK * C * C)),
            full((PK * C * C, PK * C)),
        ],
        out_specs=[
            pl.BlockSpec((BLKP, PK * C), lambda i: (i, 0)),
            full((NROWS, PK * C)),
        ],
        out_shape=[
            jax.ShapeDtypeStruct((EROWS, PK * C), jnp.float32),
            jax.ShapeDtypeStruct((NROWS, PK * C), jnp.float32),
        ],
    )(hs_pack, e_pack, ew8, b8, h_pack, root8, bias8,
      jnp.asarray(_REP8), jnp.asarray(_SUM8))


def kernel(h, e, edge_index, edge_w, edge_b, root, bias):
    src = edge_index[0].reshape(NW, NCH, CHUNK)
    dst = edge_index[1].reshape(NS, NCHS, CHUNK)
    h_src = _gather_k(h, src)
    m_pack, ob_pack = _messages(
        h_src.reshape(EROWS, PK * C),
        e.reshape(EROWS, PK * C),
        edge_w, edge_b,
        h.reshape(NROWS, PK * C),
        root, bias,
    )
    return _scatter_k(
        m_pack.reshape(N_EDGES, C), dst, ob_pack.reshape(N_NODES, C)
    )


# speedup vs baseline: 6.2880x; 1.0396x over previous
import functools

import jax
import jax.numpy as jnp
import numpy as np
from jax import lax
from jax.experimental import pallas as pl
from jax.experimental.pallas import tpu as pltpu
from jax.experimental.pallas import tpu_sc as plsc

N_NODES = 10000
N_EDGES = 160000
C = 16

NC = 2
NS = 16
NW = NC * NS
CHUNK = 100
EPW = N_EDGES // NW
NCH = EPW // CHUNK
EPT = N_EDGES // NS
NCHS = EPT // CHUNK
HALF_E = EPT // 2
HALF = NCHS // 2
STRIPE = 640
LAST_STRIPE = N_NODES - STRIPE * (NS - 1)

PK = 8
EROWS = N_EDGES // PK
NROWS = N_NODES // PK
_REP = np.repeat(np.eye(C, dtype=np.float32), C, axis=1)
_SUM = np.tile(np.eye(C, dtype=np.float32), (C, 1))
_REP8 = np.kron(np.eye(PK, dtype=np.float32), _REP)
_SUM8 = np.kron(np.eye(PK, dtype=np.float32), _SUM)

_mesh = plsc.VectorSubcoreMesh(core_axis_name="c", subcore_axis_name="s")
_sc_params = pltpu.CompilerParams(use_tc_tiling_on_sc=False, needs_layout_passes=False)


TCH = 1000
TROUNDS = EPW // TCH


@functools.partial(
    pl.kernel,
    out_type=[
        jax.ShapeDtypeStruct((N_EDGES, C), jnp.float32),
        jax.ShapeDtypeStruct((N_EDGES, C), jnp.float32),
    ],
    mesh=_mesh,
    scratch_types=[
        pltpu.VMEM((NCH, CHUNK), jnp.int32),
        pltpu.VMEM((EPW, C), jnp.float32),
        pltpu.VMEM((C, TCH), jnp.float32),
        pltpu.VMEM((TCH, C), jnp.float32),
        pltpu.SemaphoreType.DMA,
    ],
    compiler_params=_sc_params,
)
def _gather_k(h_hbm, idx_hbm, et_hbm, out_hbm, ep_hbm, idx_v, rows_v, et_v,
              ep_v, sem):
    wid = lax.axis_index("s") * NC + lax.axis_index("c")
    pltpu.sync_copy(idx_hbm.at[wid], idx_v)
    G = 10

    def body(g, carry):
        j0 = g * G
        descs = [
            pltpu.async_copy(
                h_hbm.at[idx_v.at[j0 + j]],
                rows_v.at[pl.ds((j0 + j) * CHUNK, CHUNK)],
                sem,
            )
            for j in range(G)
        ]
        for d in descs:
            d.wait()
        return carry

    lax.fori_loop(0, NCH // G, body, 0)
    pltpu.sync_copy(rows_v, out_hbm.at[pl.ds(wid * EPW, EPW)])

    lanes = lax.iota(jnp.int32, C)

    def tround(q, carry):
        base = wid * EPW + q * TCH
        pltpu.sync_copy(et_hbm.at[:, pl.ds(base, TCH)], et_v)

        def tbody(k, c2):
            ep_v[k] = plsc.load_gather(et_v, [lanes, jnp.full((C,), k, jnp.int32)])
            return c2

        lax.fori_loop(0, TCH, tbody, 0, unroll=4)
        pltpu.sync_copy(ep_v, ep_hbm.at[pl.ds(base, TCH)])
        return carry

    lax.fori_loop(0, TROUNDS, tround, 0)


@functools.partial(
    pl.kernel,
    out_type=jax.ShapeDtypeStruct((N_NODES, C), jnp.float32),
    mesh=_mesh,
    scratch_types=[
        pltpu.VMEM((HALF, CHUNK), jnp.int32),
        pltpu.VMEM((HALF_E, C), jnp.float32),
        pltpu.VMEM_SHARED((N_NODES, C), jnp.float32),
        pltpu.SemaphoreType.DMA,
    ],
    compiler_params=_sc_params,
)
def _scatter_k(m_hbm, dst_hbm, base_hbm, out_hbm, idx_v, m_v, acc, sem):
    sid = lax.axis_index("s")
    cid = lax.axis_index("c")

    @pl.when(cid == 0)
    def _():
        row0 = sid * STRIPE

        @pl.when(sid < NS - 1)
        def _():
            pltpu.sync_copy(base_hbm.at[pl.ds(row0, STRIPE)],
                            acc.at[pl.ds(row0, STRIPE)])

        @pl.when(sid == NS - 1)
        def _():
            pltpu.sync_copy(base_hbm.at[pl.ds(row0, LAST_STRIPE)],
                            acc.at[pl.ds(row0, LAST_STRIPE)])

        plsc.subcore_barrier()

        def half(p):
            pltpu.sync_copy(dst_hbm.at[sid, pl.ds(p * HALF, HALF)], idx_v)
            pltpu.sync_copy(
                m_hbm.at[pl.ds(sid * EPT + p * HALF_E, HALF_E)], m_v
            )

            G = 10

            def body(g, carry):
                j0 = g * G
                descs = [
                    pltpu.async_copy(
                        m_v.at[pl.ds((j0 + j) * CHUNK, CHUNK)],
                        acc.at[idx_v.at[j0 + j]],
                        sem,
                        add=True,
                    )
                    for j in range(G)
                ]
                for d in descs:
                    d.wait()
                return carry

            lax.fori_loop(0, HALF // G, body, 0)

        half(0)
        half(1)
        plsc.subcore_barrier()

        @pl.when(sid < NS - 1)
        def _():
            pltpu.sync_copy(acc.at[pl.ds(row0, STRIPE)],
                            out_hbm.at[pl.ds(row0, STRIPE)])

        @pl.when(sid == NS - 1)
        def _():
            pltpu.sync_copy(acc.at[pl.ds(row0, LAST_STRIPE)],
                            out_hbm.at[pl.ds(row0, LAST_STRIPE)])


def _messages_body(hsp_ref, ep_ref, ew8_ref, b8_ref, hp_ref, root8_ref,
                   bias8_ref, rep8_ref, sum8_ref, m_ref, ob_ref):
    w = jnp.dot(ep_ref[...], ew8_ref[...], preferred_element_type=jnp.float32) + b8_ref[...]
    hr = jnp.dot(hsp_ref[...], rep8_ref[...], preferred_element_type=jnp.float32)
    m_ref[...] = jnp.dot(hr * w, sum8_ref[...], preferred_element_type=jnp.float32)

    @pl.when(pl.program_id(0) == 0)
    def _():
        ob_ref[...] = (
            jnp.dot(hp_ref[...], root8_ref[...], preferred_element_type=jnp.float32)
            + bias8_ref[...]
        )


def _messages(hs_pack, e_pack, edge_w, edge_b, h_pack, root, bias):
    BLKP = 2000
    grid = (EROWS // BLKP,)
    full = lambda shape: pl.BlockSpec(shape, lambda i: tuple(0 for _ in shape))
    ew8 = jnp.kron(jnp.eye(PK, dtype=jnp.float32), edge_w)
    b8 = jnp.tile(edge_b, PK).reshape(1, PK * C * C)
    root8 = jnp.kron(jnp.eye(PK, dtype=jnp.float32), root)
    bias8 = jnp.tile(bias, PK).reshape(1, PK * C)
    return pl.pallas_call(
        _messages_body,
        grid=grid,
        in_specs=[
            pl.BlockSpec((BLKP, PK * C), lambda i: (i, 0)),
            pl.BlockSpec((BLKP, PK * C), lambda i: (i, 0)),
            full((PK * C, PK * C * C)),
            full((1, PK * C * C)),
            full((NROWS, PK * C)),
            full((PK * C, PK * C)),
            full((1, PK * C)),
            full((PK * C, PK * C * C)),
            full((PK * C * C, PK * C)),
        ],
        out_specs=[
            pl.BlockSpec((BLKP, PK * C), lambda i: (i, 0)),
            full((NROWS, PK * C)),
        ],
        out_shape=[
            jax.ShapeDtypeStruct((EROWS, PK * C), jnp.float32),
            jax.ShapeDtypeStruct((NROWS, PK * C), jnp.float32),
        ],
    )(hs_pack, e_pack, ew8, b8, h_pack, root8, bias8,
      jnp.asarray(_REP8), jnp.asarray(_SUM8))


def kernel(h, e, edge_index, edge_w, edge_b, root, bias):
    src = edge_index[0].reshape(NW, NCH, CHUNK)
    dst = edge_index[1].reshape(NS, NCHS, CHUNK)
    h_src, e_pack = _gather_k(h, src, e.T)
    m_pack, ob_pack = _messages(
        h_src.reshape(EROWS, PK * C),
        e_pack.reshape(EROWS, PK * C),
        edge_w, edge_b,
        h.reshape(NROWS, PK * C),
        root, bias,
    )
    return _scatter_k(
        m_pack.reshape(N_EDGES, C), dst, ob_pack.reshape(N_NODES, C)
    )
